# baseline (speedup 1.0000x reference)
import functools

import jax
import jax.numpy as jnp
from jax import lax
from jax.experimental import pallas as pl
from jax.experimental.pallas import tpu as pltpu
from jax.experimental.pallas import tpu_sc as plsc

N = 10000
F = 128
E = 320000
NC = 2
NS = 16
NW = NC * NS
CH = 80
SUB = 25
NSUP = 5
NCH = SUB * NSUP
E_PW = NCH * CH
NBUF = 2
ROWS_PT = 640
N_PAD = NS * ROWS_PT


_MESH = plsc.VectorSubcoreMesh(core_axis_name="c", subcore_axis_name="s")


@functools.partial(
    pl.kernel,
    out_type=jax.ShapeDtypeStruct((NC, N_PAD, F), jnp.float32),
    mesh=_MESH,
    scratch_types=[
        pltpu.VMEM((SUB * CH,), jnp.int32),
        pltpu.VMEM((SUB, CH), jnp.int32),
        pltpu.VMEM((CH, F), jnp.float32),
        pltpu.VMEM((CH, F), jnp.float32),
        pltpu.VMEM_SHARED((N_PAD, F), jnp.float32),
        pltpu.SemaphoreType.DMA,
        pltpu.SemaphoreType.DMA,
        pltpu.SemaphoreType.DMA,
    ],
)
def _scatter_gather(edges_hbm, h_hbm, zeros_hbm, out_hbm,
                    src_win, dst_win, rows0, rows1, accum,
                    gsem0, gsem1, zsem):
    c = lax.axis_index("c")
    s = lax.axis_index("s")
    wid = c * NS + s
    base = s * ROWS_PT
    bufs = (rows0, rows1)
    gsems = (gsem0, gsem1)

    def wait_gather(b):
        pltpu.make_async_copy(h_hbm.at[src_win.at[pl.ds(0, CH)]],
                              bufs[b], gsems[b]).wait()

    def stage_idx(sup):
        pltpu.sync_copy(
            edges_hbm.at[pl.ds(wid * E_PW + sup * SUB * CH, SUB * CH)],
            src_win)
        for j in range(SUB):
            pltpu.async_copy(
                edges_hbm.at[pl.ds(E + wid * E_PW + (sup * SUB + j) * CH,
                                   CH)],
                dst_win.at[j], zsem)
        for j in range(SUB):
            pltpu.make_async_copy(edges_hbm.at[pl.ds(0, CH)],
                                  dst_win.at[0], zsem).wait()

    pltpu.sync_copy(zeros_hbm, accum.at[pl.ds(base, ROWS_PT)])
    stage_idx(0)
    plsc.subcore_barrier()

    for sup in range(NSUP):
        if sup > 0:
            stage_idx(sup)

        for b in range(NBUF):
            pltpu.async_copy(h_hbm.at[src_win.at[pl.ds(b * CH, CH)]],
                             bufs[b], gsems[b])

        def chunk_step(it, carry):
            g = it * NBUF
            for b in range(NBUF):
                j = g + b
                wait_gather(b)
                pltpu.sync_copy(bufs[b], accum.at[dst_win.at[j]],
                                add=True)
                pltpu.async_copy(
                    h_hbm.at[src_win.at[pl.ds((j + NBUF) * CH, CH)]],
                    bufs[b], gsems[b])
            return carry

        lax.fori_loop(0, (SUB - NBUF) // NBUF, chunk_step, 0)

        for j in range(((SUB - NBUF) // NBUF) * NBUF, SUB):
            b = j % NBUF
            wait_gather(b)
            pltpu.sync_copy(bufs[b], accum.at[dst_win.at[j]], add=True)
            if j + NBUF < SUB:
                pltpu.async_copy(
                    h_hbm.at[src_win.at[pl.ds((j + NBUF) * CH, CH)]],
                    bufs[b], gsems[b])

    plsc.subcore_barrier()

    pltpu.sync_copy(accum.at[pl.ds(base, ROWS_PT)],
                    out_hbm.at[c, pl.ds(base, ROWS_PT)])


def _fin_body(p_ref, w_ref, b_ref, o_ref):
    agg = p_ref[0] + p_ref[1]
    o_ref[...] = jnp.maximum(
        jnp.dot(agg, w_ref[...], preferred_element_type=jnp.float32)
        + b_ref[...], 0.0)


def _finalize(partials, W, b):
    return pl.pallas_call(
        _fin_body,
        grid=(10,),
        in_specs=[
            pl.BlockSpec((NC, N // 10, F), lambda i: (0, i, 0)),
            pl.BlockSpec((F, F), lambda i: (0, 0)),
            pl.BlockSpec((1, F), lambda i: (0, 0)),
        ],
        out_specs=pl.BlockSpec((N // 10, F), lambda i: (i, 0)),
        out_shape=jax.ShapeDtypeStruct((N, F), jnp.float32),
    )(partials, W, b.reshape(1, F))


def kernel(h, edge_index, W, b):
    edges = edge_index.astype(jnp.int32).reshape(-1)
    zeros = jnp.zeros((ROWS_PT, F), jnp.float32)

    partials = _scatter_gather(edges, h, zeros)
    return _finalize(partials, W, b)


# speedup vs baseline: 1.1209x; 1.1209x over previous
import functools

import jax
import jax.numpy as jnp
from jax import lax
from jax.experimental import pallas as pl
from jax.experimental.pallas import tpu as pltpu
from jax.experimental.pallas import tpu_sc as plsc

N = 10000
F = 128
E = 320000
NC = 2
NS = 16
NW = NC * NS
CH = 128
NCHT = E // CH
CPW = 80
LAST = NCHT - 31 * CPW
SUB = 40
NBUF = 2
ROWS_PT = 640
N_PAD = NS * ROWS_PT


_MESH = plsc.VectorSubcoreMesh(core_axis_name="c", subcore_axis_name="s")


@functools.partial(
    pl.kernel,
    out_type=jax.ShapeDtypeStruct((NC, N_PAD, F), jnp.float32),
    mesh=_MESH,
    scratch_types=[
        pltpu.VMEM((SUB, CH), jnp.int32),
        pltpu.VMEM((SUB + 8, CH), jnp.int32),
        pltpu.VMEM((CH, F), jnp.float32),
        pltpu.VMEM((CH, F), jnp.float32),
        pltpu.VMEM_SHARED((N_PAD, F), jnp.float32),
        pltpu.SemaphoreType.DMA,
        pltpu.SemaphoreType.DMA,
    ],
)
def _scatter_gather(edges_hbm, h_hbm, zeros_hbm, out_hbm,
                    src_win, dst_win, rows0, rows1, accum,
                    gsem0, gsem1):
    c = lax.axis_index("c")
    s = lax.axis_index("s")
    wid = c * NS + s
    base = s * ROWS_PT
    bufs = (rows0, rows1)
    gsems = (gsem0, gsem1)

    def wait_gather(b):
        pltpu.make_async_copy(h_hbm.at[src_win.at[0]],
                              bufs[b], gsems[b]).wait()

    def process(row0, nchunks, ssize, dsize):
        row0 = pl.multiple_of(row0, 8)
        pltpu.sync_copy(edges_hbm.at[pl.ds(row0, ssize)],
                        src_win.at[pl.ds(0, ssize)])
        pltpu.sync_copy(edges_hbm.at[pl.ds(NCHT - 4 + row0, dsize)],
                        dst_win.at[pl.ds(0, dsize)])

        for b in range(NBUF):
            pltpu.async_copy(h_hbm.at[src_win.at[b]], bufs[b], gsems[b])

        def chunk_step(it, carry):
            g = it * NBUF
            for b in range(NBUF):
                j = g + b
                wait_gather(b)
                pltpu.sync_copy(bufs[b], accum.at[dst_win.at[j + 4]],
                                add=True)
                pltpu.async_copy(h_hbm.at[src_win.at[j + NBUF]],
                                 bufs[b], gsems[b])
            return carry

        lax.fori_loop(0, (nchunks - NBUF) // NBUF, chunk_step, 0)

        for b in range(NBUF):
            j = nchunks - NBUF + b
            wait_gather(b)
            pltpu.sync_copy(bufs[b], accum.at[dst_win.at[j + 4]],
                            add=True)

    pltpu.sync_copy(zeros_hbm, accum.at[pl.ds(base, ROWS_PT)])
    plsc.subcore_barrier()

    @pl.when(wid < 31)
    def _():
        process(wid * CPW, SUB, SUB, SUB + 8)
        process(wid * CPW + SUB, SUB, SUB, SUB + 8)

    @pl.when(wid == 31)
    def _():
        process(31 * CPW, LAST, LAST + 4, LAST + 4)

    plsc.subcore_barrier()

    pltpu.sync_copy(accum.at[pl.ds(base, ROWS_PT)],
                    out_hbm.at[c, pl.ds(base, ROWS_PT)])


def _fin_body(p_ref, w_ref, b_ref, o_ref):
    agg = p_ref[0] + p_ref[1]
    o_ref[...] = jnp.maximum(
        jnp.dot(agg, w_ref[...], preferred_element_type=jnp.float32)
        + b_ref[...], 0.0)


def _finalize(partials, W, b):
    return pl.pallas_call(
        _fin_body,
        grid=(10,),
        in_specs=[
            pl.BlockSpec((NC, N // 10, F), lambda i: (0, i, 0)),
            pl.BlockSpec((F, F), lambda i: (0, 0)),
            pl.BlockSpec((1, F), lambda i: (0, 0)),
        ],
        out_specs=pl.BlockSpec((N // 10, F), lambda i: (i, 0)),
        out_shape=jax.ShapeDtypeStruct((N, F), jnp.float32),
    )(partials, W, b.reshape(1, F))


def kernel(h, edge_index, W, b):
    edges = edge_index.astype(jnp.int32).reshape(2 * NCHT, CH)
    zeros = jnp.zeros((ROWS_PT, F), jnp.float32)

    partials = _scatter_gather(edges, h, zeros)
    return _finalize(partials, W, b)


# speedup vs baseline: 1.1240x; 1.0028x over previous
import functools

import jax
import jax.numpy as jnp
from jax import lax
from jax.experimental import pallas as pl
from jax.experimental.pallas import tpu as pltpu
from jax.experimental.pallas import tpu_sc as plsc

N = 10000
F = 128
E = 320000
NC = 2
NS = 16
NW = NC * NS
E_PW = E // NW
CH = 128
SUPE = 5000
NSUP = 2
SUB = SUPE // CH
TAIL = SUPE - SUB * CH
NBUF = 2
ROWS_PT = 640
N_PAD = NS * ROWS_PT


_MESH = plsc.VectorSubcoreMesh(core_axis_name="c", subcore_axis_name="s")


@functools.partial(
    pl.kernel,
    out_type=jax.ShapeDtypeStruct((NC, N_PAD, F), jnp.float32),
    mesh=_MESH,
    scratch_types=[
        pltpu.VMEM((SUPE,), jnp.int32),
        pltpu.VMEM((SUPE,), jnp.int32),
        pltpu.VMEM((CH, F), jnp.float32),
        pltpu.VMEM((CH, F), jnp.float32),
        pltpu.VMEM_SHARED((N_PAD, F), jnp.float32),
        pltpu.SemaphoreType.DMA,
        pltpu.SemaphoreType.DMA,
    ],
)
def _scatter_gather(edges_hbm, h_hbm, zeros_hbm, out_hbm,
                    src_win, dst_win, rows0, rows1, accum,
                    gsem0, gsem1):
    c = lax.axis_index("c")
    s = lax.axis_index("s")
    wid = c * NS + s
    base = s * ROWS_PT
    bufs = (rows0, rows1)
    gsems = (gsem0, gsem1)

    def wait_gather(b):
        pltpu.make_async_copy(h_hbm.at[src_win.at[pl.ds(0, CH)]],
                              bufs[b], gsems[b]).wait()

    pltpu.sync_copy(zeros_hbm, accum.at[pl.ds(base, ROWS_PT)])
    plsc.subcore_barrier()

    for sup in range(NSUP):
        off = wid * E_PW + sup * SUPE
        pltpu.sync_copy(edges_hbm.at[pl.ds(off, SUPE)], src_win)
        pltpu.sync_copy(edges_hbm.at[pl.ds(E + off, SUPE)], dst_win)

        for b in range(NBUF):
            pltpu.async_copy(h_hbm.at[src_win.at[pl.ds(b * CH, CH)]],
                             bufs[b], gsems[b])

        def chunk_step(it, carry):
            g = it * NBUF
            for b in range(NBUF):
                j = g + b
                wait_gather(b)
                pltpu.sync_copy(
                    bufs[b],
                    accum.at[dst_win.at[pl.ds(j * CH, CH)]],
                    add=True)
                pltpu.async_copy(
                    h_hbm.at[src_win.at[pl.ds((j + NBUF) * CH, CH)]],
                    bufs[b], gsems[b])
            return carry

        lax.fori_loop(0, (SUB - NBUF) // NBUF, chunk_step, 0)

        for j in range(((SUB - NBUF) // NBUF) * NBUF, SUB):
            b = j % NBUF
            wait_gather(b)
            pltpu.sync_copy(bufs[b],
                            accum.at[dst_win.at[pl.ds(j * CH, CH)]],
                            add=True)
            if j + NBUF < SUB:
                pltpu.async_copy(
                    h_hbm.at[src_win.at[pl.ds((j + NBUF) * CH, CH)]],
                    bufs[b], gsems[b])

        tcp = pltpu.async_copy(
            h_hbm.at[src_win.at[pl.ds(SUB * CH, TAIL)]],
            rows0.at[pl.ds(0, TAIL)], gsem0)
        tcp.wait()
        pltpu.sync_copy(rows0.at[pl.ds(0, TAIL)],
                        accum.at[dst_win.at[pl.ds(SUB * CH, TAIL)]],
                        add=True)

    plsc.subcore_barrier()

    pltpu.sync_copy(accum.at[pl.ds(base, ROWS_PT)],
                    out_hbm.at[c, pl.ds(base, ROWS_PT)])


def _fin_body(p_ref, w_ref, b_ref, o_ref):
    agg = p_ref[0] + p_ref[1]
    o_ref[...] = jnp.maximum(
        jnp.dot(agg, w_ref[...], preferred_element_type=jnp.float32)
        + b_ref[...], 0.0)


def _finalize(partials, W, b):
    return pl.pallas_call(
        _fin_body,
        grid=(10,),
        in_specs=[
            pl.BlockSpec((NC, N // 10, F), lambda i: (0, i, 0)),
            pl.BlockSpec((F, F), lambda i: (0, 0)),
            pl.BlockSpec((1, F), lambda i: (0, 0)),
        ],
        out_specs=pl.BlockSpec((N // 10, F), lambda i: (i, 0)),
        out_shape=jax.ShapeDtypeStruct((N, F), jnp.float32),
    )(partials, W, b.reshape(1, F))


def kernel(h, edge_index, W, b):
    edges = edge_index.astype(jnp.int32).reshape(-1)
    zeros = jnp.zeros((ROWS_PT, F), jnp.float32)

    partials = _scatter_gather(edges, h, zeros)
    return _finalize(partials, W, b)


# speedup vs baseline: 1.1578x; 1.0301x over previous
import functools

import jax
import jax.numpy as jnp
from jax import lax
from jax.experimental import pallas as pl
from jax.experimental.pallas import tpu as pltpu
from jax.experimental.pallas import tpu_sc as plsc

N = 10000
F = 128
E = 320000
NC = 2
NS = 16
NW = NC * NS
E_PW = E // NW
CH = 128
SUPE = 5000
NSUP = 2
SUB = SUPE // CH
TAIL = SUPE - SUB * CH
NBUF = 2
ROWS_PT = 640
N_PAD = NS * ROWS_PT


_MESH = plsc.VectorSubcoreMesh(core_axis_name="c", subcore_axis_name="s")


@functools.partial(
    pl.kernel,
    out_type=jax.ShapeDtypeStruct((NC, N_PAD, F), jnp.float32),
    mesh=_MESH,
    scratch_types=[
        pltpu.VMEM((SUPE,), jnp.int32),
        pltpu.VMEM((SUPE,), jnp.int32),
        pltpu.VMEM((CH, F), jnp.float32),
        pltpu.VMEM((CH, F), jnp.float32),
        pltpu.VMEM_SHARED((N_PAD, F), jnp.float32),
        pltpu.SemaphoreType.DMA,
        pltpu.SemaphoreType.DMA,
        pltpu.SemaphoreType.DMA,
    ],
)
def _scatter_gather(edges_hbm, h_hbm, zeros_hbm, out_hbm,
                    src_win, dst_win, rows0, rows1, accum,
                    gsem0, gsem1, zsem):
    c = lax.axis_index("c")
    s = lax.axis_index("s")
    wid = c * NS + s
    base = s * ROWS_PT
    bufs = (rows0, rows1)
    gsems = (gsem0, gsem1)

    def wait_gather(b):
        pltpu.make_async_copy(h_hbm.at[src_win.at[pl.ds(0, CH)]],
                              bufs[b], gsems[b]).wait()

    zcp = pltpu.async_copy(zeros_hbm, accum.at[pl.ds(base, ROWS_PT)], zsem)
    off0 = wid * E_PW
    pltpu.sync_copy(edges_hbm.at[pl.ds(off0, SUPE)], src_win)
    pltpu.sync_copy(edges_hbm.at[pl.ds(E + off0, SUPE)], dst_win)
    zcp.wait()
    plsc.subcore_barrier()

    for sup in range(NSUP):
        if sup > 0:
            off = wid * E_PW + sup * SUPE
            pltpu.sync_copy(edges_hbm.at[pl.ds(off, SUPE)], src_win)
            pltpu.sync_copy(edges_hbm.at[pl.ds(E + off, SUPE)], dst_win)

        for b in range(NBUF):
            pltpu.async_copy(h_hbm.at[src_win.at[pl.ds(b * CH, CH)]],
                             bufs[b], gsems[b])

        def chunk_step(it, carry):
            g = it * NBUF
            for b in range(NBUF):
                j = g + b
                wait_gather(b)
                pltpu.sync_copy(
                    bufs[b],
                    accum.at[dst_win.at[pl.ds(j * CH, CH)]],
                    add=True)
                pltpu.async_copy(
                    h_hbm.at[src_win.at[pl.ds((j + NBUF) * CH, CH)]],
                    bufs[b], gsems[b])
            return carry

        lax.fori_loop(0, (SUB - NBUF) // NBUF, chunk_step, 0)

        for j in range(((SUB - NBUF) // NBUF) * NBUF, SUB):
            b = j % NBUF
            wait_gather(b)
            pltpu.sync_copy(bufs[b],
                            accum.at[dst_win.at[pl.ds(j * CH, CH)]],
                            add=True)
            if j + NBUF < SUB:
                pltpu.async_copy(
                    h_hbm.at[src_win.at[pl.ds((j + NBUF) * CH, CH)]],
                    bufs[b], gsems[b])

        tcp = pltpu.async_copy(
            h_hbm.at[src_win.at[pl.ds(SUB * CH, TAIL)]],
            rows0.at[pl.ds(0, TAIL)], gsem0)
        tcp.wait()
        pltpu.sync_copy(rows0.at[pl.ds(0, TAIL)],
                        accum.at[dst_win.at[pl.ds(SUB * CH, TAIL)]],
                        add=True)

    plsc.subcore_barrier()

    pltpu.sync_copy(accum.at[pl.ds(base, ROWS_PT)],
                    out_hbm.at[c, pl.ds(base, ROWS_PT)])


def _fin_body(p_ref, w_ref, b_ref, o_ref):
    agg = p_ref[0] + p_ref[1]
    o_ref[...] = jnp.maximum(
        jnp.dot(agg, w_ref[...], preferred_element_type=jnp.float32)
        + b_ref[...], 0.0)


def _finalize(partials, W, b):
    return pl.pallas_call(
        _fin_body,
        grid=(5,),
        in_specs=[
            pl.BlockSpec((NC, N // 5, F), lambda i: (0, i, 0)),
            pl.BlockSpec((F, F), lambda i: (0, 0)),
            pl.BlockSpec((1, F), lambda i: (0, 0)),
        ],
        out_specs=pl.BlockSpec((N // 5, F), lambda i: (i, 0)),
        out_shape=jax.ShapeDtypeStruct((N, F), jnp.float32),
    )(partials, W, b.reshape(1, F))


def kernel(h, edge_index, W, b):
    edges = edge_index.astype(jnp.int32).reshape(-1)
    zeros = jnp.zeros((ROWS_PT, F), jnp.float32)

    partials = _scatter_gather(edges, h, zeros)
    return _finalize(partials, W, b)
